# baseline (speedup 1.0000x reference)
import functools
import math

import jax
import jax.numpy as jnp
from jax import lax
from jax.experimental import pallas as pl
from jax.experimental.pallas import tpu as pltpu
from jax.experimental.pallas import tpu_sc as plsc

COARSE_K = 64
FINE_K = 16
GRP = 128
NEG = -3.0e38

_INTERPRET = False


def _router_kernel(x_ref, w_ref, logits_ref, gmax_ref, *, nk, nb):
    i = pl.program_id(0)
    acc = jnp.dot(x_ref[...], w_ref[...], preferred_element_type=jnp.float32)
    col = jax.lax.broadcasted_iota(jnp.int32, acc.shape, 1) + i * nb
    acc = jnp.where(col < nk, acc, NEG)
    S = acc.shape[0]
    logits_ref[...] = acc.reshape(S // 8, 8, nb // GRP, GRP).transpose(
        0, 2, 1, 3).reshape(S // 8, (nb // GRP) * 8, GRP)
    g = jnp.max(acc.reshape(S, nb // GRP, GRP), axis=-1)
    gmax_ref[...] = g[None]


def _select_kernel(gmax_ref, x_ref, wenc_ref, ids_ref, vals_ref, q_ref):
    q_ref[...] = jnp.dot(x_ref[...], wenc_ref[...],
                         preferred_element_type=jnp.float32)
    v0 = gmax_ref[...]
    NG, T = v0.shape
    giota = jax.lax.broadcasted_iota(jnp.int32, (NG, T), 0)
    riota = jax.lax.broadcasted_iota(jnp.int32, (COARSE_K, T), 0)

    def body(j, carry):
        v, ids, vals = carry
        m = jnp.max(v, axis=0)
        hit = v == m[None, :]
        idx = jnp.min(jnp.where(hit, giota, jnp.int32(NG)), axis=0)
        ids = jnp.where(riota == j, idx[None, :], ids)
        vals = jnp.where(riota == j, m[None, :], vals)
        v = jnp.where(giota == idx[None, :], NEG, v)
        return v, ids, vals

    _, ids, vals = jax.lax.fori_loop(
        0, COARSE_K, body,
        (v0, jnp.zeros((COARSE_K, T), jnp.int32),
         jnp.full((COARSE_K, T), NEG, jnp.float32)))
    ids_ref[...] = ids
    vals_ref[...] = vals


def kernel(x, W_router, W_enc, K_all, V_all):
    B, S, D = x.shape
    NK = W_router.shape[1]
    KR = W_enc.shape[1]
    x2 = x.reshape(S, D)

    NB = 1024
    nblk = (NK + NB - 1) // NB
    NKP = nblk * NB
    NG = NKP // GRP

    logits, gmax = pl.pallas_call(
        functools.partial(_router_kernel, nk=NK, nb=NB),
        grid=(nblk,),
        in_specs=[pl.BlockSpec((S, D), lambda i: (0, 0)),
                  pl.BlockSpec((D, NB), lambda i: (0, i))],
        out_specs=[pl.BlockSpec((S // 8, (NB // GRP) * 8, GRP),
                                lambda i: (0, i, 0)),
                   pl.BlockSpec((1, S, NB // GRP), lambda i: (i, 0, 0))],
        out_shape=[jax.ShapeDtypeStruct((S // 8, NG * 8, GRP), jnp.float32),
                   jax.ShapeDtypeStruct((nblk, S, NB // GRP), jnp.float32)],
        interpret=_INTERPRET,
    )(x2, W_router)

    gmaxT = gmax.transpose(0, 2, 1).reshape(NG, S)

    TT = 128
    idsT, valsT, q = pl.pallas_call(
        _select_kernel,
        grid=(S // TT,),
        in_specs=[pl.BlockSpec((NG, TT), lambda i: (0, i)),
                  pl.BlockSpec((TT, D), lambda i: (i, 0)),
                  pl.BlockSpec((D, KR), lambda i: (0, 0))],
        out_specs=[pl.BlockSpec((COARSE_K, TT), lambda i: (0, i)),
                   pl.BlockSpec((COARSE_K, TT), lambda i: (0, i)),
                   pl.BlockSpec((TT, KR), lambda i: (i, 0))],
        out_shape=[jax.ShapeDtypeStruct((COARSE_K, S), jnp.int32),
                   jax.ShapeDtypeStruct((COARSE_K, S), jnp.float32),
                   jax.ShapeDtypeStruct((S, KR), jnp.float32)],
        interpret=_INTERPRET,
    )(gmaxT, x2, W_enc)
    ids = idsT.T
    vals = valsT.T

    rows = logits.reshape(S * NG, GRP)
    out = _sc_tail(S, D, KR, NG)(rows, ids, vals, q, K_all, V_all)
    return out.reshape(B, S, D)


def _sc_tail(S, D, KR, NG):
    NW = 32
    TPW = S // NW
    BIG = 1 << 30
    mesh = plsc.VectorSubcoreMesh(core_axis_name="c", subcore_axis_name="s")

    @functools.partial(
        pl.kernel,
        out_type=jax.ShapeDtypeStruct((S, D), jnp.float32),
        mesh=mesh,
        compiler_params=pltpu.CompilerParams(needs_layout_passes=False),
        scratch_types=[
            pltpu.VMEM((TPW, COARSE_K), jnp.int32),
            pltpu.VMEM((TPW, COARSE_K), jnp.float32),
            pltpu.VMEM((TPW, KR), jnp.float32),
            pltpu.VMEM((COARSE_K,), jnp.int32),
            pltpu.VMEM((COARSE_K,), jnp.int32),
            pltpu.VMEM((COARSE_K, GRP), jnp.float32),
            pltpu.VMEM((COARSE_K, GRP), jnp.float32),
            pltpu.VMEM((COARSE_K,), jnp.int32),
            pltpu.VMEM((COARSE_K, KR), jnp.float32),
            pltpu.VMEM((FINE_K,), jnp.int32),
            pltpu.VMEM((FINE_K, D), jnp.float32),
            pltpu.VMEM((D,), jnp.float32),
            pltpu.VMEM((D,), jnp.float32),
            pltpu.SemaphoreType.DMA,
            pltpu.SemaphoreType.DMA,
            pltpu.SemaphoreType.DMA,
            pltpu.SemaphoreType.DMA,
            pltpu.SemaphoreType.DMA,
            pltpu.SemaphoreType.DMA,
        ],
    )
    def body(rows_hbm, ids_hbm, vals_hbm, q_hbm, kall_hbm, vall_hbm, out_hbm,
             ids_v, vals_v, q_v, gidx_a, gidx_b, grp_a, grp_b, cand_v, kbuf_v,
             fsel_v, vbuf_v, outrow_a, outrow_b,
             sg_a, sg_b, sk, sv, so_a, so_b):
        wid = lax.axis_index("c") * 16 + lax.axis_index("s")
        base = wid * TPW
        pltpu.sync_copy(ids_hbm.at[pl.ds(base, TPW)], ids_v)
        pltpu.sync_copy(vals_hbm.at[pl.ds(base, TPW)], vals_v)
        pltpu.sync_copy(q_hbm.at[pl.ds(base, TPW)], q_v)
        iota = lax.broadcasted_iota(jnp.int32, (16,), 0)
        inv_sqrt = 1.0 / math.sqrt(KR)
        zeros_f = jnp.zeros((16,), jnp.float32)
        zeros_i = jnp.zeros((16,), jnp.int32)

        def _bf16round(v):
            u = plsc.bitcast(v, jnp.int32)
            r = (u + 0x7FFF + ((u >> 16) & 1)) & (-65536)
            return plsc.bitcast(r, jnp.float32)

        def _vmax(v):
            return plsc.cummax(v)[15]

        def _vmin(v):
            return -plsc.cummax(-v)[15]

        def _vsum(v):
            return plsc.cumsum(v)[15]

        def _argmax4(chunks):
            m = _vmax(jnp.maximum(jnp.maximum(chunks[0], chunks[1]),
                                  jnp.maximum(chunks[2], chunks[3])))
            o = _vmin(jnp.minimum(
                jnp.minimum(jnp.where(chunks[0] == m, iota, BIG),
                            jnp.where(chunks[1] == m, iota + 16, BIG)),
                jnp.minimum(jnp.where(chunks[2] == m, iota + 32, BIG),
                            jnp.where(chunks[3] == m, iota + 48, BIG))))
            return m, o

        def _pick4(chunks, o, other):
            return _vmin(jnp.minimum(
                jnp.minimum(jnp.where(iota == o, chunks[0], other),
                            jnp.where(iota + 16 == o, chunks[1], other)),
                jnp.minimum(jnp.where(iota + 32 == o, chunks[2], other),
                            jnp.where(iota + 48 == o, chunks[3], other))))

        def _issue_grp(i, gidx_v, grp_v, sg):
            t = base + i
            rbase = (t // 8) * (NG * 8) + t % 8
            for c4 in range(4):
                gidx_v[pl.ds(c4 * 16, 16)] = (
                    ids_v[i, pl.ds(c4 * 16, 16)] * 8 + rbase)
            pltpu.async_copy(rows_hbm.at[gidx_v], grp_v, sg)

        def _token(i, gidx_v, grp_v, sg, outrow_v, so, has_prev_out):
            t = base + i
            pltpu.make_async_copy(rows_hbm.at[gidx_v], grp_v, sg).wait()
            cmx0 = [vals_v[i, pl.ds(c4 * 16, 16)] for c4 in range(4)]

            def extract(j, carry):
                cmx, cnd = list(carry[0]), list(carry[1])
                m, g = _argmax4(cmx)
                rvs = [grp_v[g, pl.ds(kk * 16, 16)] for kk in range(8)]
                whs = [jnp.where(rvs[kk] == m, iota + kk * 16, BIG)
                       for kk in range(8)]
                ot = jnp.minimum(jnp.minimum(jnp.minimum(whs[0], whs[1]),
                                             jnp.minimum(whs[2], whs[3])),
                                 jnp.minimum(jnp.minimum(whs[4], whs[5]),
                                             jnp.minimum(whs[6], whs[7])))
                o = _vmin(ot)
                cval = g * GRP + o
                nrs = [jnp.where(iota + kk * 16 == o, NEG, rvs[kk])
                       for kk in range(8)]
                for kk in range(8):
                    grp_v[g, pl.ds(kk * 16, 16)] = nrs[kk]
                nmt = jnp.maximum(jnp.maximum(jnp.maximum(nrs[0], nrs[1]),
                                              jnp.maximum(nrs[2], nrs[3])),
                                  jnp.maximum(jnp.maximum(nrs[4], nrs[5]),
                                              jnp.maximum(nrs[6], nrs[7])))
                nm = _vmax(nmt)
                for c4 in range(4):
                    cnd[c4] = jnp.where(iota + c4 * 16 == j, cval, cnd[c4])
                    cmx[c4] = jnp.where(iota + c4 * 16 == g, nm, cmx[c4])
                return tuple(cmx), tuple(cnd)

            _, cndp = lax.fori_loop(
                0, COARSE_K, extract,
                (tuple(cmx0), (zeros_i,) * 4))
            ivec = jnp.full((16,), i, jnp.int32)
            cnd = []
            for c4 in range(4):
                gid = plsc.load_gather(ids_v, [ivec, cndp[c4] // GRP])
                cnd.append(gid * GRP + cndp[c4] % GRP)
                cand_v[pl.ds(c4 * 16, 16)] = cnd[c4]

            pltpu.async_copy(kall_hbm.at[cand_v], kbuf_v, sk).wait()
            qv = [_bf16round(q_v[i, pl.ds(kk * 16, 16)])
                  for kk in range(KR // 16)]

            scs = []
            for c4 in range(4):
                def fine(c16, acc, c4=c4):
                    c = c4 * 16 + c16
                    a = qv[0] * _bf16round(kbuf_v[c, pl.ds(0, 16)])
                    for kk in range(1, KR // 16):
                        a = a + qv[kk] * _bf16round(
                            kbuf_v[c, pl.ds(kk * 16, 16)])
                    return jnp.where(iota == c16, _vsum(a) * inv_sqrt, acc)
                scs.append(lax.fori_loop(0, 16, fine, zeros_f))

            def pick(j, carry):
                s, fsel, wv = list(carry[0]), carry[1], carry[2]
                m, o = _argmax4(s)
                cval = _pick4(cnd, o, BIG)
                fsel = jnp.where(iota == j, cval, fsel)
                wv = jnp.where(iota == j, m, wv)
                for c4 in range(4):
                    s[c4] = jnp.where(iota + c4 * 16 == o, NEG, s[c4])
                return tuple(s), fsel, wv

            _, fsel, wv = lax.fori_loop(
                0, FINE_K, pick, (tuple(scs), zeros_i, zeros_f))
            fsel_v[...] = fsel
            vcp = pltpu.async_copy(vall_hbm.at[fsel_v], vbuf_v, sv)

            e = jnp.exp(wv - _vmax(wv))
            w = e / _vsum(e)

            vcp.wait()
            ws = [w[c] for c in range(FINE_K)]

            @pl.when(has_prev_out)
            def _():
                pltpu.make_async_copy(outrow_v, out_hbm.at[t], so).wait()

            def wsum(ch, _):
                acc = ws[0] * vbuf_v[0, pl.ds(ch * 16, 16)]
                for c in range(1, FINE_K):
                    acc = acc + ws[c] * vbuf_v[c, pl.ds(ch * 16, 16)]
                outrow_v[pl.ds(ch * 16, 16)] = acc
                return 0

            lax.fori_loop(0, D // 16, wsum, 0)
            pltpu.async_copy(outrow_v, out_hbm.at[t], so)

        _issue_grp(0, gidx_a, grp_a, sg_a)

        def jbody(j, _):
            i0 = 2 * j
            i1 = 2 * j + 1
            _issue_grp(i1, gidx_b, grp_b, sg_b)
            _token(i0, gidx_a, grp_a, sg_a, outrow_a, so_a, j > 0)
            _issue_grp(jnp.minimum(i0 + 2, TPW - 1), gidx_a, grp_a, sg_a)
            _token(i1, gidx_b, grp_b, sg_b, outrow_b, so_b, j > 0)
            return 0

        lax.fori_loop(0, TPW // 2, jbody, 0)
        pltpu.make_async_copy(rows_hbm.at[gidx_a], grp_a, sg_a).wait()
        pltpu.make_async_copy(outrow_a, out_hbm.at[base], so_a).wait()
        pltpu.make_async_copy(outrow_b, out_hbm.at[base], so_b).wait()

    return body


# speedup vs baseline: 1.1185x; 1.1185x over previous
import functools
import math

import jax
import jax.numpy as jnp
from jax import lax
from jax.experimental import pallas as pl
from jax.experimental.pallas import tpu as pltpu
from jax.experimental.pallas import tpu_sc as plsc

COARSE_K = 64
FINE_K = 16
GRP = 128
NEG = -3.0e38

_INTERPRET = False


def _router_kernel(x_ref, w_ref, logits_ref, gmax_ref, *, nk, nb):
    i = pl.program_id(0)
    acc = jnp.dot(x_ref[...], w_ref[...], preferred_element_type=jnp.float32)
    col = jax.lax.broadcasted_iota(jnp.int32, acc.shape, 1) + i * nb
    acc = jnp.where(col < nk, acc, NEG)
    S = acc.shape[0]
    logits_ref[...] = acc.reshape(S // 8, 8, nb // GRP, GRP).transpose(
        0, 2, 1, 3).reshape(S // 8, (nb // GRP) * 8, GRP)
    g = jnp.max(acc.reshape(S, nb // GRP, GRP), axis=-1)
    gmax_ref[...] = g[None]


def _select_kernel(gmax_ref, x_ref, wenc_ref, ids_ref, vals_ref, q_ref):
    q_ref[...] = jnp.dot(x_ref[...], wenc_ref[...],
                         preferred_element_type=jnp.float32)
    v0 = gmax_ref[...]
    NG, T = v0.shape
    giota = jax.lax.broadcasted_iota(jnp.int32, (NG, T), 0)
    riota = jax.lax.broadcasted_iota(jnp.int32, (COARSE_K, T), 0)

    def body(j, carry):
        v, ids, vals = carry
        m = jnp.max(v, axis=0)
        hit = v == m[None, :]
        idx = jnp.min(jnp.where(hit, giota, jnp.int32(NG)), axis=0)
        ids = jnp.where(riota == j, idx[None, :], ids)
        vals = jnp.where(riota == j, m[None, :], vals)
        v = jnp.where(giota == idx[None, :], NEG, v)
        return v, ids, vals

    _, ids, vals = jax.lax.fori_loop(
        0, COARSE_K, body,
        (v0, jnp.zeros((COARSE_K, T), jnp.int32),
         jnp.full((COARSE_K, T), NEG, jnp.float32)))
    ids_ref[...] = ids
    vals_ref[...] = vals


def kernel(x, W_router, W_enc, K_all, V_all):
    B, S, D = x.shape
    NK = W_router.shape[1]
    KR = W_enc.shape[1]
    x2 = x.reshape(S, D)

    NB = 1024
    nblk = (NK + NB - 1) // NB
    NKP = nblk * NB
    NG = NKP // GRP

    logits, gmax = pl.pallas_call(
        functools.partial(_router_kernel, nk=NK, nb=NB),
        grid=(nblk,),
        in_specs=[pl.BlockSpec((S, D), lambda i: (0, 0)),
                  pl.BlockSpec((D, NB), lambda i: (0, i))],
        out_specs=[pl.BlockSpec((S // 8, (NB // GRP) * 8, GRP),
                                lambda i: (0, i, 0)),
                   pl.BlockSpec((1, S, NB // GRP), lambda i: (i, 0, 0))],
        out_shape=[jax.ShapeDtypeStruct((S // 8, NG * 8, GRP), jnp.float32),
                   jax.ShapeDtypeStruct((nblk, S, NB // GRP), jnp.float32)],
        interpret=_INTERPRET,
    )(x2, W_router)

    gmaxT = gmax.transpose(0, 2, 1).reshape(NG, S)

    TT = 128
    idsT, valsT, q = pl.pallas_call(
        _select_kernel,
        grid=(S // TT,),
        in_specs=[pl.BlockSpec((NG, TT), lambda i: (0, i)),
                  pl.BlockSpec((TT, D), lambda i: (i, 0)),
                  pl.BlockSpec((D, KR), lambda i: (0, 0))],
        out_specs=[pl.BlockSpec((COARSE_K, TT), lambda i: (0, i)),
                   pl.BlockSpec((COARSE_K, TT), lambda i: (0, i)),
                   pl.BlockSpec((TT, KR), lambda i: (i, 0))],
        out_shape=[jax.ShapeDtypeStruct((COARSE_K, S), jnp.int32),
                   jax.ShapeDtypeStruct((COARSE_K, S), jnp.float32),
                   jax.ShapeDtypeStruct((S, KR), jnp.float32)],
        interpret=_INTERPRET,
    )(gmaxT, x2, W_enc)
    ids = idsT.T
    vals = valsT.T

    rows = logits.reshape(S * NG, GRP)
    out = _sc_tail(S, D, KR, NG)(rows, ids, vals, q, K_all, V_all)
    return out.reshape(B, S, D)


def _sc_tail(S, D, KR, NG):
    NW = 32
    TPW = S // NW
    BIG = 1 << 30
    mesh = plsc.VectorSubcoreMesh(core_axis_name="c", subcore_axis_name="s")

    @functools.partial(
        pl.kernel,
        out_type=jax.ShapeDtypeStruct((S, D), jnp.float32),
        mesh=mesh,
        compiler_params=pltpu.CompilerParams(needs_layout_passes=False),
        scratch_types=[
            pltpu.VMEM((TPW, COARSE_K), jnp.int32),
            pltpu.VMEM((TPW, COARSE_K), jnp.float32),
            pltpu.VMEM((TPW, KR), jnp.float32),
            pltpu.VMEM((COARSE_K,), jnp.int32),
            pltpu.VMEM((COARSE_K,), jnp.int32),
            pltpu.VMEM((COARSE_K, GRP), jnp.float32),
            pltpu.VMEM((COARSE_K, GRP), jnp.float32),
            pltpu.VMEM((COARSE_K,), jnp.int32),
            pltpu.VMEM((COARSE_K,), jnp.int32),
            pltpu.VMEM((COARSE_K, KR), jnp.float32),
            pltpu.VMEM((COARSE_K, KR), jnp.float32),
            pltpu.VMEM((FINE_K,), jnp.int32),
            pltpu.VMEM((FINE_K,), jnp.int32),
            pltpu.VMEM((FINE_K, D), jnp.float32),
            pltpu.VMEM((FINE_K, D), jnp.float32),
            pltpu.VMEM((D,), jnp.float32),
            pltpu.VMEM((D,), jnp.float32),
            pltpu.SemaphoreType.DMA,
            pltpu.SemaphoreType.DMA,
            pltpu.SemaphoreType.DMA,
            pltpu.SemaphoreType.DMA,
            pltpu.SemaphoreType.DMA,
            pltpu.SemaphoreType.DMA,
            pltpu.SemaphoreType.DMA,
            pltpu.SemaphoreType.DMA,
        ],
    )
    def body(rows_hbm, ids_hbm, vals_hbm, q_hbm, kall_hbm, vall_hbm, out_hbm,
             ids_v, vals_v, q_v, gidx_a, gidx_b, grp_a, grp_b, cand_a, cand_b,
             kbuf_a, kbuf_b, fsel_a, fsel_b, vbuf_a, vbuf_b,
             outrow_a, outrow_b,
             sg_a, sg_b, sk_a, sk_b, sv_a, sv_b, so_a, so_b):
        wid = lax.axis_index("c") * 16 + lax.axis_index("s")
        base = wid * TPW
        pltpu.sync_copy(ids_hbm.at[pl.ds(base, TPW)], ids_v)
        pltpu.sync_copy(vals_hbm.at[pl.ds(base, TPW)], vals_v)
        pltpu.sync_copy(q_hbm.at[pl.ds(base, TPW)], q_v)
        iota = lax.broadcasted_iota(jnp.int32, (16,), 0)
        inv_sqrt = 1.0 / math.sqrt(KR)
        zeros_f = jnp.zeros((16,), jnp.float32)
        zeros_i = jnp.zeros((16,), jnp.int32)

        def _bf16round(v):
            u = plsc.bitcast(v, jnp.int32)
            r = (u + 0x7FFF + ((u >> 16) & 1)) & (-65536)
            return plsc.bitcast(r, jnp.float32)

        def _vmax(v):
            return plsc.cummax(v)[15]

        def _vmin(v):
            return -plsc.cummax(-v)[15]

        def _vsum(v):
            return plsc.cumsum(v)[15]

        def _argmax4(chunks):
            m = _vmax(jnp.maximum(jnp.maximum(chunks[0], chunks[1]),
                                  jnp.maximum(chunks[2], chunks[3])))
            o = _vmin(jnp.minimum(
                jnp.minimum(jnp.where(chunks[0] == m, iota, BIG),
                            jnp.where(chunks[1] == m, iota + 16, BIG)),
                jnp.minimum(jnp.where(chunks[2] == m, iota + 32, BIG),
                            jnp.where(chunks[3] == m, iota + 48, BIG))))
            return m, o

        def _pick4(chunks, o, other):
            return _vmin(jnp.minimum(
                jnp.minimum(jnp.where(iota == o, chunks[0], other),
                            jnp.where(iota + 16 == o, chunks[1], other)),
                jnp.minimum(jnp.where(iota + 32 == o, chunks[2], other),
                            jnp.where(iota + 48 == o, chunks[3], other))))

        def _issue_grp(i, gidx_v, grp_v, sg):
            t = base + i
            rbase = (t // 8) * (NG * 8) + t % 8
            for c4 in range(4):
                gidx_v[pl.ds(c4 * 16, 16)] = (
                    ids_v[i, pl.ds(c4 * 16, 16)] * 8 + rbase)
            pltpu.async_copy(rows_hbm.at[gidx_v], grp_v, sg)

        def _ext_step(j, carry, grp_v):
            cmx, cnd = list(carry[0]), list(carry[1])
            m, g = _argmax4(cmx)
            rvs = [grp_v[g, pl.ds(kk * 16, 16)] for kk in range(8)]
            whs = [jnp.where(rvs[kk] == m, iota + kk * 16, BIG)
                   for kk in range(8)]
            ot = jnp.minimum(jnp.minimum(jnp.minimum(whs[0], whs[1]),
                                         jnp.minimum(whs[2], whs[3])),
                             jnp.minimum(jnp.minimum(whs[4], whs[5]),
                                         jnp.minimum(whs[6], whs[7])))
            o = _vmin(ot)
            cval = g * GRP + o
            nrs = [jnp.where(iota + kk * 16 == o, NEG, rvs[kk])
                   for kk in range(8)]
            for kk in range(8):
                grp_v[g, pl.ds(kk * 16, 16)] = nrs[kk]
            nmt = jnp.maximum(jnp.maximum(jnp.maximum(nrs[0], nrs[1]),
                                          jnp.maximum(nrs[2], nrs[3])),
                              jnp.maximum(jnp.maximum(nrs[4], nrs[5]),
                                          jnp.maximum(nrs[6], nrs[7])))
            nm = _vmax(nmt)
            for c4 in range(4):
                cnd[c4] = jnp.where(iota + c4 * 16 == j, cval, cnd[c4])
                cmx[c4] = jnp.where(iota + c4 * 16 == g, nm, cmx[c4])
            return tuple(cmx), tuple(cnd)

        def _resolve(i, cndp, cand_v):
            ivec = jnp.full((16,), i, jnp.int32)
            cnd = []
            for c4 in range(4):
                gid = plsc.load_gather(ids_v, [ivec, cndp[c4] // GRP])
                cnd.append(gid * GRP + cndp[c4] % GRP)
                cand_v[pl.ds(c4 * 16, 16)] = cnd[c4]
            return cnd

        def _fine_pair(iA, iB):
            qA = [_bf16round(q_v[iA, pl.ds(kk * 16, 16)])
                  for kk in range(KR // 16)]
            qB = [_bf16round(q_v[iB, pl.ds(kk * 16, 16)])
                  for kk in range(KR // 16)]
            scsA, scsB = [], []
            for c4 in range(4):
                def fine(c16, acc, c4=c4):
                    c = c4 * 16 + c16
                    a = qA[0] * _bf16round(kbuf_a[c, pl.ds(0, 16)])
                    b = qB[0] * _bf16round(kbuf_b[c, pl.ds(0, 16)])
                    for kk in range(1, KR // 16):
                        a = a + qA[kk] * _bf16round(
                            kbuf_a[c, pl.ds(kk * 16, 16)])
                        b = b + qB[kk] * _bf16round(
                            kbuf_b[c, pl.ds(kk * 16, 16)])
                    return (jnp.where(iota == c16, _vsum(a) * inv_sqrt,
                                      acc[0]),
                            jnp.where(iota == c16, _vsum(b) * inv_sqrt,
                                      acc[1]))
                rA, rB = lax.fori_loop(0, 16, fine, (zeros_f, zeros_f))
                scsA.append(rA)
                scsB.append(rB)
            return scsA, scsB

        def _pick_step(j, carry, cnd):
            s, fsel, wv = list(carry[0]), carry[1], carry[2]
            m, o = _argmax4(s)
            cval = _pick4(cnd, o, BIG)
            fsel = jnp.where(iota == j, cval, fsel)
            wv = jnp.where(iota == j, m, wv)
            for c4 in range(4):
                s[c4] = jnp.where(iota + c4 * 16 == o, NEG, s[c4])
            return tuple(s), fsel, wv

        def _softmax(wv):
            e = jnp.exp(wv - _vmax(wv))
            return e / _vsum(e)

        def _wsum_pair(wA, wB):
            wsA = [wA[c] for c in range(FINE_K)]
            wsB = [wB[c] for c in range(FINE_K)]

            def wsum(ch, _):
                a = wsA[0] * vbuf_a[0, pl.ds(ch * 16, 16)]
                b = wsB[0] * vbuf_b[0, pl.ds(ch * 16, 16)]
                for c in range(1, FINE_K):
                    a = a + wsA[c] * vbuf_a[c, pl.ds(ch * 16, 16)]
                    b = b + wsB[c] * vbuf_b[c, pl.ds(ch * 16, 16)]
                outrow_a[pl.ds(ch * 16, 16)] = a
                outrow_b[pl.ds(ch * 16, 16)] = b
                return 0

            lax.fori_loop(0, D // 16, wsum, 0)

        _issue_grp(0, gidx_a, grp_a, sg_a)
        _issue_grp(1, gidx_b, grp_b, sg_b)

        def jbody(j, _):
            i0 = 2 * j
            i1 = 2 * j + 1
            t0 = base + i0
            t1 = base + i1
            pltpu.make_async_copy(rows_hbm.at[gidx_a], grp_a, sg_a).wait()
            pltpu.make_async_copy(rows_hbm.at[gidx_b], grp_b, sg_b).wait()

            def extract2(j2, carry):
                return (_ext_step(j2, carry[0], grp_a),
                        _ext_step(j2, carry[1], grp_b))

            cmxA = [vals_v[i0, pl.ds(c4 * 16, 16)] for c4 in range(4)]
            cmxB = [vals_v[i1, pl.ds(c4 * 16, 16)] for c4 in range(4)]
            stA, stB = lax.fori_loop(
                0, COARSE_K, extract2,
                ((tuple(cmxA), (zeros_i,) * 4),
                 (tuple(cmxB), (zeros_i,) * 4)))
            cndA = _resolve(i0, stA[1], cand_a)
            cndB = _resolve(i1, stB[1], cand_b)
            pltpu.async_copy(kall_hbm.at[cand_a], kbuf_a, sk_a)
            pltpu.async_copy(kall_hbm.at[cand_b], kbuf_b, sk_b)

            _issue_grp(jnp.minimum(i0 + 2, TPW - 1), gidx_a, grp_a, sg_a)
            _issue_grp(jnp.minimum(i1 + 2, TPW - 1), gidx_b, grp_b, sg_b)

            pltpu.make_async_copy(kall_hbm.at[cand_a], kbuf_a, sk_a).wait()
            pltpu.make_async_copy(kall_hbm.at[cand_b], kbuf_b, sk_b).wait()
            scsA, scsB = _fine_pair(i0, i1)

            def pick2(j2, carry):
                return (_pick_step(j2, carry[0], cndA),
                        _pick_step(j2, carry[1], cndB))

            pA, pB = lax.fori_loop(
                0, FINE_K, pick2,
                ((tuple(scsA), zeros_i, zeros_f),
                 (tuple(scsB), zeros_i, zeros_f)))
            fsel_a[...] = pA[1]
            fsel_b[...] = pB[1]
            pltpu.async_copy(vall_hbm.at[fsel_a], vbuf_a, sv_a)
            pltpu.async_copy(vall_hbm.at[fsel_b], vbuf_b, sv_b)
            wA = _softmax(pA[2])
            wB = _softmax(pB[2])
            pltpu.make_async_copy(vall_hbm.at[fsel_a], vbuf_a, sv_a).wait()
            pltpu.make_async_copy(vall_hbm.at[fsel_b], vbuf_b, sv_b).wait()

            @pl.when(j > 0)
            def _():
                pltpu.make_async_copy(outrow_a, out_hbm.at[t0], so_a).wait()
                pltpu.make_async_copy(outrow_b, out_hbm.at[t1], so_b).wait()

            _wsum_pair(wA, wB)
            pltpu.async_copy(outrow_a, out_hbm.at[t0], so_a)
            pltpu.async_copy(outrow_b, out_hbm.at[t1], so_b)
            return 0

        lax.fori_loop(0, TPW // 2, jbody, 0)
        pltpu.make_async_copy(rows_hbm.at[gidx_a], grp_a, sg_a).wait()
        pltpu.make_async_copy(rows_hbm.at[gidx_b], grp_b, sg_b).wait()
        pltpu.make_async_copy(outrow_a, out_hbm.at[base], so_a).wait()
        pltpu.make_async_copy(outrow_b, out_hbm.at[base], so_b).wait()

    return body


# speedup vs baseline: 1.1189x; 1.0003x over previous
import functools
import math

import jax
import jax.numpy as jnp
from jax import lax
from jax.experimental import pallas as pl
from jax.experimental.pallas import tpu as pltpu
from jax.experimental.pallas import tpu_sc as plsc

COARSE_K = 64
FINE_K = 16
GRP = 128
NEG = -3.0e38


def _router_kernel(x_ref, w_ref, logits_ref, gmax_ref, *, nk, nb):
    i = pl.program_id(0)
    acc = jnp.dot(x_ref[...], w_ref[...], preferred_element_type=jnp.float32)
    col = jax.lax.broadcasted_iota(jnp.int32, acc.shape, 1) + i * nb
    acc = jnp.where(col < nk, acc, NEG)
    S = acc.shape[0]
    logits_ref[...] = acc.reshape(S // 8, 8, nb // GRP, GRP).transpose(
        0, 2, 1, 3).reshape(S // 8, (nb // GRP) * 8, GRP)
    g = jnp.max(acc.reshape(S, nb // GRP, GRP), axis=-1)
    gmax_ref[...] = g[None]


def _select_kernel(gmax_ref, x_ref, wenc_ref, ids_ref, vals_ref, q_ref):
    q_ref[...] = jnp.dot(x_ref[...], wenc_ref[...],
                         preferred_element_type=jnp.float32)
    v0 = gmax_ref[...]
    NG, T = v0.shape
    giota = jax.lax.broadcasted_iota(jnp.int32, (NG, T), 0)
    riota = jax.lax.broadcasted_iota(jnp.int32, (COARSE_K, T), 0)

    def body(j, carry):
        v, ids, vals = carry
        m = jnp.max(v, axis=0)
        hit = v == m[None, :]
        idx = jnp.min(jnp.where(hit, giota, jnp.int32(NG)), axis=0)
        ids = jnp.where(riota == j, idx[None, :], ids)
        vals = jnp.where(riota == j, m[None, :], vals)
        v = jnp.where(giota == idx[None, :], NEG, v)
        return v, ids, vals

    _, ids, vals = jax.lax.fori_loop(
        0, COARSE_K, body,
        (v0, jnp.zeros((COARSE_K, T), jnp.int32),
         jnp.full((COARSE_K, T), NEG, jnp.float32)))
    ids_ref[...] = ids
    vals_ref[...] = vals


def kernel(x, W_router, W_enc, K_all, V_all):
    B, S, D = x.shape
    NK = W_router.shape[1]
    KR = W_enc.shape[1]
    x2 = x.reshape(S, D)

    NB = 1024
    nblk = (NK + NB - 1) // NB
    NKP = nblk * NB
    NG = NKP // GRP

    logits, gmax = pl.pallas_call(
        functools.partial(_router_kernel, nk=NK, nb=NB),
        grid=(nblk,),
        in_specs=[pl.BlockSpec((S, D), lambda i: (0, 0)),
                  pl.BlockSpec((D, NB), lambda i: (0, i))],
        out_specs=[pl.BlockSpec((S // 8, (NB // GRP) * 8, GRP),
                                lambda i: (0, i, 0)),
                   pl.BlockSpec((1, S, NB // GRP), lambda i: (i, 0, 0))],
        out_shape=[jax.ShapeDtypeStruct((S // 8, NG * 8, GRP), jnp.float32),
                   jax.ShapeDtypeStruct((nblk, S, NB // GRP), jnp.float32)],
    )(x2, W_router)

    gmaxT = gmax.transpose(0, 2, 1).reshape(NG, S)

    TT = 128
    idsT, valsT, q = pl.pallas_call(
        _select_kernel,
        grid=(S // TT,),
        in_specs=[pl.BlockSpec((NG, TT), lambda i: (0, i)),
                  pl.BlockSpec((TT, D), lambda i: (i, 0)),
                  pl.BlockSpec((D, KR), lambda i: (0, 0))],
        out_specs=[pl.BlockSpec((COARSE_K, TT), lambda i: (0, i)),
                   pl.BlockSpec((COARSE_K, TT), lambda i: (0, i)),
                   pl.BlockSpec((TT, KR), lambda i: (i, 0))],
        out_shape=[jax.ShapeDtypeStruct((COARSE_K, S), jnp.int32),
                   jax.ShapeDtypeStruct((COARSE_K, S), jnp.float32),
                   jax.ShapeDtypeStruct((S, KR), jnp.float32)],
    )(gmaxT, x2, W_enc)
    ids = idsT.T
    vals = valsT.T

    rows = logits.reshape(S * NG, GRP)
    out = _sc_tail(S, D, KR, NG)(rows, ids, vals, q, K_all, V_all)
    return out.reshape(B, S, D)


def _sc_tail(S, D, KR, NG):
    NW = 32
    TPW = S // NW
    BIG = 1 << 30
    mesh = plsc.VectorSubcoreMesh(core_axis_name="c", subcore_axis_name="s")

    @functools.partial(
        pl.kernel,
        out_type=jax.ShapeDtypeStruct((S, D), jnp.float32),
        mesh=mesh,
        compiler_params=pltpu.CompilerParams(needs_layout_passes=False),
        scratch_types=[
            pltpu.VMEM((TPW, COARSE_K), jnp.int32),
            pltpu.VMEM((TPW, COARSE_K), jnp.float32),
            pltpu.VMEM((TPW, KR), jnp.float32),
            pltpu.VMEM((COARSE_K,), jnp.int32),
            pltpu.VMEM((COARSE_K,), jnp.int32),
            pltpu.VMEM((COARSE_K, GRP), jnp.float32),
            pltpu.VMEM((COARSE_K, GRP), jnp.float32),
            pltpu.VMEM((COARSE_K,), jnp.int32),
            pltpu.VMEM((COARSE_K,), jnp.int32),
            pltpu.VMEM((COARSE_K, KR), jnp.float32),
            pltpu.VMEM((COARSE_K, KR), jnp.float32),
            pltpu.VMEM((FINE_K,), jnp.int32),
            pltpu.VMEM((FINE_K,), jnp.int32),
            pltpu.VMEM((FINE_K, D), jnp.float32),
            pltpu.VMEM((FINE_K, D), jnp.float32),
            pltpu.VMEM((D,), jnp.float32),
            pltpu.VMEM((D,), jnp.float32),
            pltpu.SemaphoreType.DMA,
            pltpu.SemaphoreType.DMA,
            pltpu.SemaphoreType.DMA,
            pltpu.SemaphoreType.DMA,
            pltpu.SemaphoreType.DMA,
            pltpu.SemaphoreType.DMA,
            pltpu.SemaphoreType.DMA,
            pltpu.SemaphoreType.DMA,
        ],
    )
    def body(rows_hbm, ids_hbm, vals_hbm, q_hbm, kall_hbm, vall_hbm, out_hbm,
             ids_v, vals_v, q_v, gidx_a, gidx_b, grp_a, grp_b, cand_a, cand_b,
             kbuf_a, kbuf_b, fsel_a, fsel_b, vbuf_a, vbuf_b,
             outrow_a, outrow_b,
             sg_a, sg_b, sk_a, sk_b, sv_a, sv_b, so_a, so_b):
        wid = lax.axis_index("c") * 16 + lax.axis_index("s")
        base = wid * TPW
        pltpu.sync_copy(ids_hbm.at[pl.ds(base, TPW)], ids_v)
        pltpu.sync_copy(vals_hbm.at[pl.ds(base, TPW)], vals_v)
        pltpu.sync_copy(q_hbm.at[pl.ds(base, TPW)], q_v)
        iota = lax.broadcasted_iota(jnp.int32, (16,), 0)
        inv_sqrt = 1.0 / math.sqrt(KR)
        zeros_f = jnp.zeros((16,), jnp.float32)
        zeros_i = jnp.zeros((16,), jnp.int32)

        def _bf16round(v):
            u = plsc.bitcast(v, jnp.int32)
            r = (u + 0x7FFF + ((u >> 16) & 1)) & (-65536)
            return plsc.bitcast(r, jnp.float32)

        def _vmax(v):
            return plsc.cummax(v)[15]

        def _vmin(v):
            return -plsc.cummax(-v)[15]

        def _vsum(v):
            return plsc.cumsum(v)[15]

        def _argmax4(chunks):
            m = _vmax(jnp.maximum(jnp.maximum(chunks[0], chunks[1]),
                                  jnp.maximum(chunks[2], chunks[3])))
            o = _vmin(jnp.minimum(
                jnp.minimum(jnp.where(chunks[0] == m, iota, BIG),
                            jnp.where(chunks[1] == m, iota + 16, BIG)),
                jnp.minimum(jnp.where(chunks[2] == m, iota + 32, BIG),
                            jnp.where(chunks[3] == m, iota + 48, BIG))))
            return m, o

        def _pick4(chunks, o, other):
            return _vmin(jnp.minimum(
                jnp.minimum(jnp.where(iota == o, chunks[0], other),
                            jnp.where(iota + 16 == o, chunks[1], other)),
                jnp.minimum(jnp.where(iota + 32 == o, chunks[2], other),
                            jnp.where(iota + 48 == o, chunks[3], other))))

        def _issue_grp(i, gidx_v, grp_v, sg):
            t = base + i
            rbase = (t // 8) * (NG * 8) + t % 8
            for c4 in range(4):
                gidx_v[pl.ds(c4 * 16, 16)] = (
                    ids_v[i, pl.ds(c4 * 16, 16)] * 8 + rbase)
            pltpu.async_copy(rows_hbm.at[gidx_v], grp_v, sg)

        def _ext_step(j, carry, grp_v):
            cmx, cnd = list(carry[0]), list(carry[1])
            m, g = _argmax4(cmx)
            rvs = [grp_v[g, pl.ds(kk * 16, 16)] for kk in range(8)]
            whs = [jnp.where(rvs[kk] == m, iota + kk * 16, BIG)
                   for kk in range(8)]
            ot = jnp.minimum(jnp.minimum(jnp.minimum(whs[0], whs[1]),
                                         jnp.minimum(whs[2], whs[3])),
                             jnp.minimum(jnp.minimum(whs[4], whs[5]),
                                         jnp.minimum(whs[6], whs[7])))
            o = _vmin(ot)
            cval = g * GRP + o
            nrs = [jnp.where(iota + kk * 16 == o, NEG, rvs[kk])
                   for kk in range(8)]
            for kk in range(8):
                grp_v[g, pl.ds(kk * 16, 16)] = nrs[kk]
            nmt = jnp.maximum(jnp.maximum(jnp.maximum(nrs[0], nrs[1]),
                                          jnp.maximum(nrs[2], nrs[3])),
                              jnp.maximum(jnp.maximum(nrs[4], nrs[5]),
                                          jnp.maximum(nrs[6], nrs[7])))
            nm = _vmax(nmt)
            for c4 in range(4):
                cnd[c4] = jnp.where(iota + c4 * 16 == j, cval, cnd[c4])
                cmx[c4] = jnp.where(iota + c4 * 16 == g, nm, cmx[c4])
            return tuple(cmx), tuple(cnd)

        def _resolve(i, cndp, cand_v):
            ivec = jnp.full((16,), i, jnp.int32)
            cnd = []
            for c4 in range(4):
                gid = plsc.load_gather(ids_v, [ivec, cndp[c4] // GRP])
                cnd.append(gid * GRP + cndp[c4] % GRP)
                cand_v[pl.ds(c4 * 16, 16)] = cnd[c4]
            return cnd

        def _fine_pair(iA, iB):
            qA = [_bf16round(q_v[iA, pl.ds(kk * 16, 16)])
                  for kk in range(KR // 16)]
            qB = [_bf16round(q_v[iB, pl.ds(kk * 16, 16)])
                  for kk in range(KR // 16)]
            scsA, scsB = [], []
            for c4 in range(4):
                def fine(c16, acc, c4=c4):
                    c = c4 * 16 + c16
                    a = qA[0] * _bf16round(kbuf_a[c, pl.ds(0, 16)])
                    b = qB[0] * _bf16round(kbuf_b[c, pl.ds(0, 16)])
                    for kk in range(1, KR // 16):
                        a = a + qA[kk] * _bf16round(
                            kbuf_a[c, pl.ds(kk * 16, 16)])
                        b = b + qB[kk] * _bf16round(
                            kbuf_b[c, pl.ds(kk * 16, 16)])
                    return (jnp.where(iota == c16, _vsum(a) * inv_sqrt,
                                      acc[0]),
                            jnp.where(iota == c16, _vsum(b) * inv_sqrt,
                                      acc[1]))
                rA, rB = lax.fori_loop(0, 16, fine, (zeros_f, zeros_f))
                scsA.append(rA)
                scsB.append(rB)
            return scsA, scsB

        def _pick_step(j, carry, cnd):
            s, fsel, wv = list(carry[0]), carry[1], carry[2]
            m, o = _argmax4(s)
            cval = _pick4(cnd, o, BIG)
            fsel = jnp.where(iota == j, cval, fsel)
            wv = jnp.where(iota == j, m, wv)
            for c4 in range(4):
                s[c4] = jnp.where(iota + c4 * 16 == o, NEG, s[c4])
            return tuple(s), fsel, wv

        def _softmax(wv):
            e = jnp.exp(wv - _vmax(wv))
            return e / _vsum(e)

        def _wsum_pair(wA, wB):
            wsA = [wA[c] for c in range(FINE_K)]
            wsB = [wB[c] for c in range(FINE_K)]

            def wsum(ch, _):
                a = wsA[0] * vbuf_a[0, pl.ds(ch * 16, 16)]
                b = wsB[0] * vbuf_b[0, pl.ds(ch * 16, 16)]
                for c in range(1, FINE_K):
                    a = a + wsA[c] * vbuf_a[c, pl.ds(ch * 16, 16)]
                    b = b + wsB[c] * vbuf_b[c, pl.ds(ch * 16, 16)]
                outrow_a[pl.ds(ch * 16, 16)] = a
                outrow_b[pl.ds(ch * 16, 16)] = b
                return 0

            lax.fori_loop(0, D // 16, wsum, 0)

        _issue_grp(0, gidx_a, grp_a, sg_a)
        _issue_grp(1, gidx_b, grp_b, sg_b)

        def jbody(j, _):
            i0 = 2 * j
            i1 = 2 * j + 1
            t0 = base + i0
            t1 = base + i1
            pltpu.make_async_copy(rows_hbm.at[gidx_a], grp_a, sg_a).wait()
            pltpu.make_async_copy(rows_hbm.at[gidx_b], grp_b, sg_b).wait()

            def extract2(j2, carry):
                return (_ext_step(j2, carry[0], grp_a),
                        _ext_step(j2, carry[1], grp_b))

            cmxA = [vals_v[i0, pl.ds(c4 * 16, 16)] for c4 in range(4)]
            cmxB = [vals_v[i1, pl.ds(c4 * 16, 16)] for c4 in range(4)]
            stA, stB = lax.fori_loop(
                0, COARSE_K, extract2,
                ((tuple(cmxA), (zeros_i,) * 4),
                 (tuple(cmxB), (zeros_i,) * 4)))
            cndA = _resolve(i0, stA[1], cand_a)
            cndB = _resolve(i1, stB[1], cand_b)
            pltpu.async_copy(kall_hbm.at[cand_a], kbuf_a, sk_a)
            pltpu.async_copy(kall_hbm.at[cand_b], kbuf_b, sk_b)

            _issue_grp(jnp.minimum(i0 + 2, TPW - 1), gidx_a, grp_a, sg_a)
            _issue_grp(jnp.minimum(i1 + 2, TPW - 1), gidx_b, grp_b, sg_b)

            pltpu.make_async_copy(kall_hbm.at[cand_a], kbuf_a, sk_a).wait()
            pltpu.make_async_copy(kall_hbm.at[cand_b], kbuf_b, sk_b).wait()
            scsA, scsB = _fine_pair(i0, i1)

            def pick2(j2, carry):
                return (_pick_step(j2, carry[0], cndA),
                        _pick_step(j2, carry[1], cndB))

            pA, pB = lax.fori_loop(
                0, FINE_K, pick2,
                ((tuple(scsA), zeros_i, zeros_f),
                 (tuple(scsB), zeros_i, zeros_f)))
            fsel_a[...] = pA[1]
            fsel_b[...] = pB[1]
            pltpu.async_copy(vall_hbm.at[fsel_a], vbuf_a, sv_a)
            pltpu.async_copy(vall_hbm.at[fsel_b], vbuf_b, sv_b)
            wA = _softmax(pA[2])
            wB = _softmax(pB[2])
            pltpu.make_async_copy(vall_hbm.at[fsel_a], vbuf_a, sv_a).wait()
            pltpu.make_async_copy(vall_hbm.at[fsel_b], vbuf_b, sv_b).wait()

            @pl.when(j > 0)
            def _():
                pltpu.make_async_copy(outrow_a, out_hbm.at[t0], so_a).wait()
                pltpu.make_async_copy(outrow_b, out_hbm.at[t1], so_b).wait()

            _wsum_pair(wA, wB)
            pltpu.async_copy(outrow_a, out_hbm.at[t0], so_a)
            pltpu.async_copy(outrow_b, out_hbm.at[t1], so_b)
            return 0

        lax.fori_loop(0, TPW // 2, jbody, 0)
        pltpu.make_async_copy(rows_hbm.at[gidx_a], grp_a, sg_a).wait()
        pltpu.make_async_copy(rows_hbm.at[gidx_b], grp_b, sg_b).wait()
        pltpu.make_async_copy(outrow_a, out_hbm.at[base], so_a).wait()
        pltpu.make_async_copy(outrow_b, out_hbm.at[base], so_b).wait()

    return body
